# baseline (speedup 1.0000x reference)
import functools

import jax
import jax.numpy as jnp
from jax import lax
from jax.experimental import pallas as pl
from jax.experimental.pallas import tpu as pltpu
from jax.experimental.pallas import tpu_sc as plsc

B = 16384
EMB = 64
NC = 2
NS = 16
NW = NC * NS
BPW = B // NW
CHUNK = 128
NCHUNK = BPW // CHUNK

_mesh = plsc.VectorSubcoreMesh(core_axis_name="c", subcore_axis_name="s")


@functools.partial(
    pl.kernel,
    out_type=jax.ShapeDtypeStruct((B,), jnp.float32),
    mesh=_mesh,
    compiler_params=pltpu.CompilerParams(needs_layout_passes=False,
                                         use_tc_tiling_on_sc=False),
    scratch_types=[
        pltpu.VMEM((NCHUNK, CHUNK), jnp.int32),
        pltpu.VMEM((NCHUNK, CHUNK), jnp.int32),
        pltpu.VMEM((NCHUNK, CHUNK, EMB), jnp.float32),
        pltpu.VMEM((NCHUNK, CHUNK, EMB), jnp.float32),
        pltpu.VMEM((NCHUNK, CHUNK), jnp.float32),
        pltpu.VMEM((NCHUNK, CHUNK), jnp.float32),
        pltpu.VMEM((BPW * 16,), jnp.float32),
        pltpu.VMEM((BPW,), jnp.float32),
        pltpu.SemaphoreType.DMA,
    ],
)
def _sc_kernel(users_hbm, items_hbm, uemb_hbm, iemb_hbm, ubias_hbm,
               ibias_hbm, out_hbm, uidx, iidx, urows, irows, ub, ib,
               part, outb, sem):
    wid = lax.axis_index("s") * NC + lax.axis_index("c")
    base = wid * BPW

    for j in range(NCHUNK):
        pltpu.sync_copy(users_hbm.at[pl.ds(base + j * CHUNK, CHUNK)],
                        uidx.at[j])
        pltpu.sync_copy(items_hbm.at[pl.ds(base + j * CHUNK, CHUNK)],
                        iidx.at[j])

    handles = []
    for j in range(NCHUNK):
        handles.append(pltpu.async_copy(uemb_hbm.at[uidx.at[j]],
                                        urows.at[j], sem))
        handles.append(pltpu.async_copy(iemb_hbm.at[iidx.at[j]],
                                        irows.at[j], sem))
        handles.append(pltpu.async_copy(ubias_hbm.at[uidx.at[j]],
                                        ub.at[j], sem))
        handles.append(pltpu.async_copy(ibias_hbm.at[iidx.at[j]],
                                        ib.at[j], sem))
    for h in handles:
        h.wait()

    for j in range(NCHUNK):
        def row_body(r, _, j=j):
            acc = (urows[j, r, pl.ds(0, 16)] * irows[j, r, pl.ds(0, 16)])
            for k in range(1, EMB // 16):
                acc = acc + (urows[j, r, pl.ds(k * 16, 16)]
                             * irows[j, r, pl.ds(k * 16, 16)])
            part[pl.ds((j * CHUNK + r) * 16, 16)] = acc
            return 0
        lax.fori_loop(0, CHUNK, row_body, 0)

    iota16 = lax.iota(jnp.int32, 16)
    for j in range(NCHUNK):
        def grp_body(g, _, j=j):
            row0 = j * CHUNK + g * 16
            vec0 = row0 * 16 + iota16 * 16
            acc = plsc.load_gather(part, [vec0])
            for l in range(1, 16):
                acc = acc + plsc.load_gather(part, [vec0 + l])
            res = acc + ub[j, pl.ds(g * 16, 16)] + ib[j, pl.ds(g * 16, 16)]
            outb[pl.ds(row0, 16)] = res
            return 0
        lax.fori_loop(0, CHUNK // 16, grp_body, 0)

    pltpu.sync_copy(outb, out_hbm.at[pl.ds(base, BPW)])


def kernel(users, items, user_emb, item_emb, user_bias, item_bias):
    return _sc_kernel(users.astype(jnp.int32), items.astype(jnp.int32),
                      user_emb, item_emb, user_bias.reshape(-1),
                      item_bias.reshape(-1))
